# baseline (speedup 1.0000x reference)
import functools

import jax
import jax.numpy as jnp
from jax import lax
from jax.experimental import pallas as pl
from jax.experimental.pallas import tpu as pltpu
from jax.experimental.pallas import tpu_sc as plsc

L = 16
NUM_WORKERS = 32
PASS_ROWS = 128
REV4 = [int("{:04b}".format(k)[::-1], 2) for k in range(L)]

_DNUMS = lax.GatherDimensionNumbers(
    offset_dims=(), collapsed_slice_dims=(0,), start_index_map=(0,))


def _shuf(v, perm):
    return lax.gather(v, perm, _DNUMS, slice_sizes=(1,),
                      mode=lax.GatherScatterMode.PROMISE_IN_BOUNDS)


def _build(B, D, NR):
    b_per_w = B // NUM_WORKERS
    n_pass = b_per_w // PASS_ROWS
    g_per_pass = PASS_ROWS // L
    n_slices = D // L

    mesh = plsc.VectorSubcoreMesh(core_axis_name="c", subcore_axis_name="s")

    @functools.partial(
        pl.kernel,
        mesh=mesh,
        out_type=jax.ShapeDtypeStruct((B,), jnp.float32),
        scratch_types=[
            pltpu.VMEM((b_per_w,), jnp.int32),
            pltpu.VMEM((b_per_w,), jnp.int32),
            pltpu.VMEM((b_per_w,), jnp.int32),
            pltpu.VMEM((2 * PASS_ROWS, D), jnp.float32),
            pltpu.VMEM((2 * PASS_ROWS, 128), jnp.float32),
            pltpu.VMEM((2 * PASS_ROWS, D), jnp.float32),
            pltpu.VMEM((b_per_w,), jnp.float32),
            pltpu.SemaphoreType.DMA,
            pltpu.SemaphoreType.DMA,
        ],
    )
    def distmult(h_hbm, r_hbm, t_hbm, ent_hbm, rel_hbm, out_hbm,
                 h_idx, r_idx, t_idx, h_rows, r_rows, t_rows, out_v,
                 sem0, sem1):
        wid = lax.axis_index("s") * 2 + lax.axis_index("c")
        base = wid * b_per_w
        sems = (sem0, sem1)

        pltpu.sync_copy(h_hbm.at[pl.ds(base, b_per_w)], h_idx)
        pltpu.sync_copy(r_hbm.at[pl.ds(base, b_per_w)], r_idx)
        pltpu.sync_copy(t_hbm.at[pl.ds(base, b_per_w)], t_idx)

        for i in range(b_per_w // L):
            sl = pl.ds(i * L, L)
            rv = r_idx[sl]
            r_idx[sl] = jnp.where(rv >= NR, rv - NR, rv)

        lane = lax.iota(jnp.int32, L)
        perms = {d: (lane ^ d).reshape(L, 1) for d in (8, 4, 2, 1)}
        masks = {d: (lane & d) == 0 for d in (8, 4, 2, 1)}

        def issue_rel(p, boff, sem):
            pltpu.async_copy(
                rel_hbm.at[r_idx.at[pl.ds(p * PASS_ROWS, PASS_ROWS)]],
                r_rows.at[pl.ds(boff, PASS_ROWS)], sem)

        def issue_group(p, g, boff, sem):
            row0 = g * L
            hv = h_idx[pl.ds(p * PASS_ROWS + row0, L)]
            tv = t_idx[pl.ds(p * PASS_ROWS + row0, L)]
            for k in range(L):
                pltpu.async_copy(
                    ent_hbm.at[pl.ds(hv[k], 1), :],
                    h_rows.at[pl.ds(boff + row0 + k, 1), :], sem)
                pltpu.async_copy(
                    ent_hbm.at[pl.ds(tv[k], 1), :],
                    t_rows.at[pl.ds(boff + row0 + k, 1), :], sem)

        def drain(boff, sem):
            dummy = ent_hbm.at[pl.ds(0, PASS_ROWS), :]
            dst = pl.ds(boff, PASS_ROWS)
            pltpu.make_async_copy(dummy, h_rows.at[dst, :], sem).wait()
            pltpu.make_async_copy(dummy, t_rows.at[dst, :], sem).wait()
            pltpu.make_async_copy(
                rel_hbm.at[pl.ds(0, PASS_ROWS), :],
                r_rows.at[dst, :], sem).wait()

        def compute_group(p, g, boff):
            row0 = g * L
            vecs = []
            for k in range(L):
                row = boff + row0 + REV4[k]
                acc = None
                for s in range(n_slices):
                    dsl = pl.ds(s * L, L)
                    prod = (h_rows[row, dsl] * r_rows[row, dsl]
                            * t_rows[row, dsl])
                    acc = prod if acc is None else acc + prod
                vecs.append(acc)
            for d in (8, 4, 2, 1):
                nxt = []
                for i in range(len(vecs) // 2):
                    u, v = vecs[2 * i], vecs[2 * i + 1]
                    m = jnp.where(masks[d], u, v)
                    n = jnp.where(masks[d], v, u)
                    nxt.append(m + _shuf(n, perms[d]))
                vecs = nxt
            out_v[pl.ds(p * PASS_ROWS + row0, L)] = vecs[0]

        def issue(p, boff, sem):
            issue_rel(p, boff, sem)

            @plsc.parallel_loop(0, g_per_pass)
            def _(g):
                issue_group(p, g, boff, sem)

        def compute(p, boff):
            @plsc.parallel_loop(0, g_per_pass)
            def _(g):
                compute_group(p, g, boff)

        n_q = n_pass // 2
        issue(0, 0, sem0)

        def qbody(q, c):
            p0 = 2 * q
            issue(p0 + 1, PASS_ROWS, sem1)
            drain(0, sem0)
            compute(p0, 0)

            @pl.when(q < n_q - 1)
            def _():
                issue(p0 + 2, 0, sem0)

            drain(PASS_ROWS, sem1)
            compute(p0 + 1, PASS_ROWS)
            return c

        lax.fori_loop(0, n_q, qbody, 0)
        pltpu.sync_copy(out_v, out_hbm.at[pl.ds(base, b_per_w)])

    return distmult


def kernel(h, r, t, ent_weight, rel_weight):
    B = h.shape[0]
    D = ent_weight.shape[1]
    NR = rel_weight.shape[0]
    rel128 = jnp.pad(rel_weight, ((0, 0), (0, 128 - D)))
    fn = _build(B, D, NR)
    return fn(h.astype(jnp.int32), r.astype(jnp.int32), t.astype(jnp.int32),
              ent_weight, rel128)


# speedup vs baseline: 1.1475x; 1.1475x over previous
import functools

import jax
import jax.numpy as jnp
from jax import lax
from jax.experimental import pallas as pl
from jax.experimental.pallas import tpu as pltpu
from jax.experimental.pallas import tpu_sc as plsc

L = 16
NUM_WORKERS = 32
PASS_ROWS = 128
REV4 = [int("{:04b}".format(k)[::-1], 2) for k in range(L)]

_DNUMS = lax.GatherDimensionNumbers(
    offset_dims=(), collapsed_slice_dims=(0,), start_index_map=(0,))


def _shuf(v, perm):
    return lax.gather(v, perm, _DNUMS, slice_sizes=(1,),
                      mode=lax.GatherScatterMode.PROMISE_IN_BOUNDS)


def _build(B, D, NR):
    b_per_w = B // NUM_WORKERS
    n_pass = b_per_w // PASS_ROWS
    g_per_pass = PASS_ROWS // L
    n_slices = D // L

    mesh = plsc.VectorSubcoreMesh(core_axis_name="c", subcore_axis_name="s")

    @functools.partial(
        pl.kernel,
        mesh=mesh,
        out_type=jax.ShapeDtypeStruct((B,), jnp.float32),
        scratch_types=[
            pltpu.VMEM((b_per_w,), jnp.int32),
            pltpu.VMEM((b_per_w,), jnp.int32),
            pltpu.VMEM((b_per_w,), jnp.int32),
            pltpu.VMEM((2 * PASS_ROWS, D), jnp.float32),
            pltpu.VMEM((2 * PASS_ROWS, 128), jnp.float32),
            pltpu.VMEM((2 * PASS_ROWS, D), jnp.float32),
            pltpu.VMEM((b_per_w,), jnp.float32),
            pltpu.SemaphoreType.DMA,
            pltpu.SemaphoreType.DMA,
        ],
    )
    def distmult(h_hbm, r_hbm, t_hbm, ent_hbm, rel_hbm, out_hbm,
                 h_idx, r_idx, t_idx, h_rows, r_rows, t_rows, out_v,
                 sem0, sem1):
        wid = lax.axis_index("s") * 2 + lax.axis_index("c")
        base = wid * b_per_w
        sems = (sem0, sem1)

        pltpu.sync_copy(h_hbm.at[pl.ds(base, b_per_w)], h_idx)
        pltpu.sync_copy(r_hbm.at[pl.ds(base, b_per_w)], r_idx)
        pltpu.sync_copy(t_hbm.at[pl.ds(base, b_per_w)], t_idx)

        for i in range(b_per_w // L):
            sl = pl.ds(i * L, L)
            rv = r_idx[sl]
            r_idx[sl] = jnp.where(rv >= NR, rv - NR, rv)

        lane = lax.iota(jnp.int32, L)
        perms = {d: (lane ^ d).reshape(L, 1) for d in (8, 4, 2, 1)}
        masks = {d: (lane & d) == 0 for d in (8, 4, 2, 1)}

        def issue_rel(p, boff, sem):
            pltpu.async_copy(
                rel_hbm.at[r_idx.at[pl.ds(p * PASS_ROWS, PASS_ROWS)]],
                r_rows.at[pl.ds(boff, PASS_ROWS)], sem)

        def issue_group(p, g, boff, sem):
            row0 = g * L
            hv = h_idx[pl.ds(p * PASS_ROWS + row0, L)]
            tv = t_idx[pl.ds(p * PASS_ROWS + row0, L)]
            for k in range(L):
                pltpu.async_copy(
                    ent_hbm.at[pl.ds(hv[k], 1), :],
                    h_rows.at[pl.ds(boff + row0 + k, 1), :], sem)
                pltpu.async_copy(
                    ent_hbm.at[pl.ds(tv[k], 1), :],
                    t_rows.at[pl.ds(boff + row0 + k, 1), :], sem)

        def drain(boff, sem):
            dummy = ent_hbm.at[pl.ds(0, PASS_ROWS), :]
            dst = pl.ds(boff, PASS_ROWS)
            pltpu.make_async_copy(dummy, h_rows.at[dst, :], sem).wait()
            pltpu.make_async_copy(dummy, t_rows.at[dst, :], sem).wait()
            pltpu.make_async_copy(
                rel_hbm.at[pl.ds(0, PASS_ROWS), :],
                r_rows.at[dst, :], sem).wait()

        def compute_group(p, g, boff):
            row0 = g * L
            vecs = []
            for k in range(L):
                row = boff + row0 + REV4[k]
                acc = None
                for s in range(n_slices):
                    dsl = pl.ds(s * L, L)
                    prod = (h_rows[row, dsl] * r_rows[row, dsl]
                            * t_rows[row, dsl])
                    acc = prod if acc is None else acc + prod
                vecs.append(acc)
            for d in (8, 4, 2, 1):
                nxt = []
                for i in range(len(vecs) // 2):
                    u, v = vecs[2 * i], vecs[2 * i + 1]
                    m = jnp.where(masks[d], u, v)
                    n = jnp.where(masks[d], v, u)
                    nxt.append(m + _shuf(n, perms[d]))
                vecs = nxt
            out_v[pl.ds(p * PASS_ROWS + row0, L)] = vecs[0]

        def issue(p, boff, sem):
            issue_rel(p, boff, sem)

            def issue_body(g, c):
                issue_group(p, g, boff, sem)
                return c

            lax.fori_loop(0, g_per_pass, issue_body, 0)

        def compute(p, boff):
            def group_body(g, c):
                compute_group(p, g, boff)
                return c

            lax.fori_loop(0, g_per_pass, group_body, 0)

        n_q = n_pass // 2
        issue(0, 0, sem0)

        def qbody(q, c):
            p0 = 2 * q
            issue(p0 + 1, PASS_ROWS, sem1)
            drain(0, sem0)
            compute(p0, 0)

            @pl.when(q < n_q - 1)
            def _():
                issue(p0 + 2, 0, sem0)

            drain(PASS_ROWS, sem1)
            compute(p0 + 1, PASS_ROWS)
            return c

        lax.fori_loop(0, n_q, qbody, 0)
        pltpu.sync_copy(out_v, out_hbm.at[pl.ds(base, b_per_w)])

    return distmult


def kernel(h, r, t, ent_weight, rel_weight):
    B = h.shape[0]
    D = ent_weight.shape[1]
    NR = rel_weight.shape[0]
    rel128 = jnp.pad(rel_weight, ((0, 0), (0, 128 - D)))
    fn = _build(B, D, NR)
    return fn(h.astype(jnp.int32), r.astype(jnp.int32), t.astype(jnp.int32),
              ent_weight, rel128)
